# Initial kernel scaffold; baseline (speedup 1.0000x reference)
import jax
import jax.numpy as jnp
from jax.experimental import pallas as pl


def kernel(atom_features, f_bonds, a2b, a2a, a_scope, W_i, b_i, W_h, b_h, W_o, b_o):
    raise NotImplementedError("write your pallas kernel here")


# speedup vs baseline: 1.3775x; 1.3775x over previous
import functools

import jax
import jax.numpy as jnp
from jax import lax
from jax.experimental import pallas as pl
from jax.experimental.pallas import tpu as pltpu
from jax.experimental.pallas import tpu_sc as plsc

ATOM_FDIM = 128
BOND_FDIM = 16
HIDDEN = 128
DEPTH = 3
N_ATOMS = 10000
MAX_NB = 32
N_MOLS = 100
MOL_SIZE = 100

NC, NS, L = 2, 16, 16
NW = NC * NS
N_PAD = 10240
APW = N_PAD // NW
CH = 4
NCHUNK = APW // CH


def _make_gsum(h, tc_tiling=True):
    mesh = plsc.VectorSubcoreMesh(core_axis_name="c", subcore_axis_name="s")
    nreg = h // L

    @functools.partial(
        pl.kernel,
        out_type=jax.ShapeDtypeStruct((N_PAD, h), jnp.float32),
        mesh=mesh,
        compiler_params=pltpu.CompilerParams(use_tc_tiling_on_sc=tc_tiling),
        scratch_types=[
            pltpu.VMEM((NCHUNK, CH * MAX_NB), jnp.int32),
            pltpu.VMEM((CH * MAX_NB, h), jnp.float32),
            pltpu.VMEM((APW, h), jnp.float32),
            pltpu.SemaphoreType.DMA,
        ],
    )
    def gsum(idx_hbm, table_hbm, out_hbm, idx_v, rows_v, acc_v, sem):
        wid = lax.axis_index("s") * NC + lax.axis_index("c")
        pltpu.sync_copy(idx_hbm.at[wid], idx_v)

        def chunk(c, carry):
            pltpu.async_copy(table_hbm.at[idx_v.at[c]], rows_v, sem).wait()
            for a in range(CH):
                r0 = a * MAX_NB

                def red(r, accs, r0=r0):
                    return tuple(accs[k] + rows_v[r0 + r, pl.ds(k * L, L)]
                                 for k in range(nreg))

                accs = tuple(rows_v[r0, pl.ds(k * L, L)] for k in range(nreg))
                accs = lax.fori_loop(1, MAX_NB, red, accs)
                for k in range(nreg):
                    acc_v[c * CH + a, pl.ds(k * L, L)] = accs[k]
            return carry

        lax.fori_loop(0, NCHUNK, chunk, 0)
        pltpu.sync_copy(acc_v, out_hbm.at[pl.ds(wid * APW, APW)])

    return gsum


_gsum_bond = _make_gsum(BOND_FDIM, tc_tiling=False)
_gsum_msg = _make_gsum(HIDDEN)


def _tc_init(af, bsum, wit, bi, whbt, bh):
    def body(af_ref, bs_ref, wi_ref, bi_ref, whb_ref, bh_ref, msg0_ref, base_ref):
        inp = jnp.dot(af_ref[...], wi_ref[...],
                      preferred_element_type=jnp.float32) + bi_ref[...]
        msg0_ref[...] = jnp.maximum(inp, 0.0)
        base_ref[...] = inp + jnp.dot(bs_ref[...], whb_ref[...],
                                      preferred_element_type=jnp.float32) + bh_ref[...]

    return pl.pallas_call(
        body,
        out_shape=[jax.ShapeDtypeStruct((N_PAD, HIDDEN), jnp.float32)] * 2,
    )(af, bsum, wit, bi, whbt, bh)


def _tc_update(s, base, whmt):
    def body(s_ref, base_ref, w_ref, out_ref):
        out_ref[...] = jnp.maximum(
            jnp.dot(s_ref[...], w_ref[...],
                    preferred_element_type=jnp.float32) + base_ref[...], 0.0)

    return pl.pallas_call(
        body,
        out_shape=jax.ShapeDtypeStruct((N_PAD, HIDDEN), jnp.float32),
    )(s, base, whmt)


def _tc_out(af, s, woat, womt, bo):
    def body(af_ref, s_ref, wa_ref, wm_ref, bo_ref, out_ref):
        hid = jnp.dot(af_ref[...], wa_ref[...], preferred_element_type=jnp.float32)
        hid = hid + jnp.dot(s_ref[...], wm_ref[...],
                            preferred_element_type=jnp.float32) + bo_ref[...]
        hid = jnp.maximum(hid, 0.0)
        row = lax.broadcasted_iota(jnp.int32, (N_MOLS, N_PAD), 0)
        col = lax.broadcasted_iota(jnp.int32, (N_MOLS, N_PAD), 1)
        pool = jnp.where(col // MOL_SIZE == row, 1.0 / MOL_SIZE, 0.0)
        out_ref[...] = jnp.dot(pool, hid, preferred_element_type=jnp.float32)

    return pl.pallas_call(
        body,
        out_shape=jax.ShapeDtypeStruct((N_MOLS, HIDDEN), jnp.float32),
    )(af, s, woat, womt, bo)


def kernel(atom_features, f_bonds, a2b, a2a, a_scope, W_i, b_i, W_h, b_h, W_o, b_o):
    pad = N_PAD - N_ATOMS
    af_p = jnp.concatenate(
        [atom_features, jnp.zeros((pad, ATOM_FDIM), jnp.float32)], axis=0)
    a2a_p = jnp.concatenate(
        [a2a.astype(jnp.int32), jnp.zeros((pad, MAX_NB), jnp.int32)],
        axis=0).reshape(NW, NCHUNK, CH * MAX_NB)
    a2b_p = jnp.concatenate(
        [a2b.astype(jnp.int32), jnp.zeros((pad, MAX_NB), jnp.int32)],
        axis=0).reshape(NW, NCHUNK, CH * MAX_NB)

    wit = W_i.T
    whmt = W_h[:, :HIDDEN].T
    whbt = W_h[:, HIDDEN:].T
    woat = W_o[:, :ATOM_FDIM].T
    womt = W_o[:, ATOM_FDIM:].T
    bi = b_i.reshape(1, HIDDEN)
    bh = b_h.reshape(1, HIDDEN)
    bo = b_o.reshape(1, HIDDEN)

    bond_sum = _gsum_bond(a2b_p, f_bonds)
    msg, base = _tc_init(af_p, bond_sum, wit, bi, whbt, bh)
    for _ in range(DEPTH - 1):
        s = _gsum_msg(a2a_p, msg)
        msg = _tc_update(s, base, whmt)
    s = _gsum_msg(a2a_p, msg)
    return _tc_out(af_p, s, woat, womt, bo)


# speedup vs baseline: 1.4302x; 1.0383x over previous
import functools

import jax
import jax.numpy as jnp
from jax import lax
from jax.experimental import pallas as pl
from jax.experimental.pallas import tpu as pltpu
from jax.experimental.pallas import tpu_sc as plsc

ATOM_FDIM = 128
BOND_FDIM = 16
HIDDEN = 128
DEPTH = 3
N_ATOMS = 10000
MAX_NB = 32
N_MOLS = 100
MOL_SIZE = 100

NC, NS, L = 2, 16, 16
NW = NC * NS
N_PAD = 10240
APW = N_PAD // NW
CH = 4
NCHUNK = APW // CH


NBUF = 4
RU = 8


def _make_gsum(h, tc_tiling=True):
    mesh = plsc.VectorSubcoreMesh(core_axis_name="c", subcore_axis_name="s")
    nreg = h // L

    @functools.partial(
        pl.kernel,
        out_type=jax.ShapeDtypeStruct((N_PAD, h), jnp.float32),
        mesh=mesh,
        compiler_params=pltpu.CompilerParams(use_tc_tiling_on_sc=tc_tiling),
        scratch_types=[
            pltpu.VMEM((NCHUNK, CH * MAX_NB), jnp.int32),
            [pltpu.VMEM((CH * MAX_NB, h), jnp.float32)] * NBUF,
            pltpu.VMEM((APW, h), jnp.float32),
            [pltpu.SemaphoreType.DMA] * NBUF,
        ],
    )
    def gsum(idx_hbm, table_hbm, out_hbm, idx_v, rows, acc_v, sems):
        wid = lax.axis_index("s") * NC + lax.axis_index("c")
        pltpu.sync_copy(idx_hbm.at[wid], idx_v)
        for b in range(NBUF):
            pltpu.async_copy(table_hbm.at[idx_v.at[b]], rows[b], sems[b])

        def group(j, carry):
            c2 = j * NBUF
            for b in range(NBUF):
                c = c2 + b
                pltpu.make_async_copy(
                    table_hbm.at[idx_v.at[c]], rows[b], sems[b]).wait()
                for a in range(CH):
                    r0 = a * MAX_NB

                    def red(ri, accs, b=b, r0=r0):
                        base = r0 + ri * RU
                        out = list(accs)
                        for rr in range(RU):
                            for k in range(nreg):
                                out[k] = out[k] + rows[b][base + rr,
                                                          pl.ds(k * L, L)]
                        return tuple(out)

                    zero = jnp.zeros((L,), jnp.float32)
                    accs = lax.fori_loop(0, MAX_NB // RU, red, (zero,) * nreg)
                    for k in range(nreg):
                        acc_v[c * CH + a, pl.ds(k * L, L)] = accs[k]
                nxt = jnp.minimum(c + NBUF, NCHUNK - 1)
                pltpu.async_copy(table_hbm.at[idx_v.at[nxt]], rows[b], sems[b])
            return carry

        lax.fori_loop(0, NCHUNK // NBUF, group, 0)
        for b in range(NBUF):
            pltpu.make_async_copy(
                table_hbm.at[idx_v.at[0]], rows[b], sems[b]).wait()
        pltpu.sync_copy(acc_v, out_hbm.at[pl.ds(wid * APW, APW)])

    return gsum


_gsum_bond = _make_gsum(BOND_FDIM, tc_tiling=False)
_gsum_msg = _make_gsum(HIDDEN)


def _tc_init(af, bsum, wit, bi, whbt, bh):
    def body(af_ref, bs_ref, wi_ref, bi_ref, whb_ref, bh_ref, msg0_ref, base_ref):
        inp = jnp.dot(af_ref[...], wi_ref[...],
                      preferred_element_type=jnp.float32) + bi_ref[...]
        msg0_ref[...] = jnp.maximum(inp, 0.0)
        base_ref[...] = inp + jnp.dot(bs_ref[...], whb_ref[...],
                                      preferred_element_type=jnp.float32) + bh_ref[...]

    return pl.pallas_call(
        body,
        out_shape=[jax.ShapeDtypeStruct((N_PAD, HIDDEN), jnp.float32)] * 2,
    )(af, bsum, wit, bi, whbt, bh)


def _tc_update(s, base, whmt):
    def body(s_ref, base_ref, w_ref, out_ref):
        out_ref[...] = jnp.maximum(
            jnp.dot(s_ref[...], w_ref[...],
                    preferred_element_type=jnp.float32) + base_ref[...], 0.0)

    return pl.pallas_call(
        body,
        out_shape=jax.ShapeDtypeStruct((N_PAD, HIDDEN), jnp.float32),
    )(s, base, whmt)


def _tc_out(af, s, woat, womt, bo):
    def body(af_ref, s_ref, wa_ref, wm_ref, bo_ref, out_ref):
        hid = jnp.dot(af_ref[...], wa_ref[...], preferred_element_type=jnp.float32)
        hid = hid + jnp.dot(s_ref[...], wm_ref[...],
                            preferred_element_type=jnp.float32) + bo_ref[...]
        hid = jnp.maximum(hid, 0.0)
        row = lax.broadcasted_iota(jnp.int32, (N_MOLS, N_PAD), 0)
        col = lax.broadcasted_iota(jnp.int32, (N_MOLS, N_PAD), 1)
        pool = jnp.where(col // MOL_SIZE == row, 1.0 / MOL_SIZE, 0.0)
        out_ref[...] = jnp.dot(pool, hid, preferred_element_type=jnp.float32)

    return pl.pallas_call(
        body,
        out_shape=jax.ShapeDtypeStruct((N_MOLS, HIDDEN), jnp.float32),
    )(af, s, woat, womt, bo)


def kernel(atom_features, f_bonds, a2b, a2a, a_scope, W_i, b_i, W_h, b_h, W_o, b_o):
    pad = N_PAD - N_ATOMS
    af_p = jnp.concatenate(
        [atom_features, jnp.zeros((pad, ATOM_FDIM), jnp.float32)], axis=0)
    a2a_p = jnp.concatenate(
        [a2a.astype(jnp.int32), jnp.zeros((pad, MAX_NB), jnp.int32)],
        axis=0).reshape(NW, NCHUNK, CH * MAX_NB)
    a2b_p = jnp.concatenate(
        [a2b.astype(jnp.int32), jnp.zeros((pad, MAX_NB), jnp.int32)],
        axis=0).reshape(NW, NCHUNK, CH * MAX_NB)

    wit = W_i.T
    whmt = W_h[:, :HIDDEN].T
    whbt = W_h[:, HIDDEN:].T
    woat = W_o[:, :ATOM_FDIM].T
    womt = W_o[:, ATOM_FDIM:].T
    bi = b_i.reshape(1, HIDDEN)
    bh = b_h.reshape(1, HIDDEN)
    bo = b_o.reshape(1, HIDDEN)

    bond_sum = _gsum_bond(a2b_p, f_bonds)
    msg, base = _tc_init(af_p, bond_sum, wit, bi, whbt, bh)
    for _ in range(DEPTH - 1):
        s = _gsum_msg(a2a_p, msg)
        msg = _tc_update(s, base, whmt)
    s = _gsum_msg(a2a_p, msg)
    return _tc_out(af_p, s, woat, womt, bo)


# speedup vs baseline: 6.4843x; 4.5339x over previous
import functools

import jax
import jax.numpy as jnp
from jax import lax
from jax.experimental import pallas as pl
from jax.experimental.pallas import tpu as pltpu
from jax.experimental.pallas import tpu_sc as plsc

ATOM_FDIM = 128
BOND_FDIM = 16
HIDDEN = 128
DEPTH = 3
N_ATOMS = 10000
MAX_NB = 32
N_MOLS = 100
MOL_SIZE = 100

NC, NS, L = 2, 16, 16
NW = NC * NS
N_PAD = 10240
APW = N_PAD // NW
NBUF = 4
RU = 8
AHALF = APW // 2


def _reduce_rows(rows_ref, nreg):
    def red(ri, accs):
        base = ri * RU
        out = list(accs)
        for rr in range(RU):
            for k in range(nreg):
                out[k] = out[k] + rows_ref[base + rr, pl.ds(k * L, L)]
        return tuple(out)

    zero = jnp.zeros((L,), jnp.float32)
    return lax.fori_loop(0, MAX_NB // RU, red, (zero,) * nreg)


def _make_gsum_hbm(h):
    mesh = plsc.VectorSubcoreMesh(core_axis_name="c", subcore_axis_name="s")
    nreg = h // L

    @functools.partial(
        pl.kernel,
        out_type=jax.ShapeDtypeStruct((N_PAD, h), jnp.float32),
        mesh=mesh,
        compiler_params=pltpu.CompilerParams(use_tc_tiling_on_sc=False),
        scratch_types=[
            pltpu.VMEM((APW, MAX_NB), jnp.int32),
            [pltpu.VMEM((MAX_NB, h), jnp.float32)] * NBUF,
            pltpu.VMEM((APW, h), jnp.float32),
            [pltpu.SemaphoreType.DMA] * NBUF,
        ],
    )
    def gsum(idx_hbm, table_hbm, out_hbm, idx_v, rows, acc_v, sems):
        wid = lax.axis_index("s") * NC + lax.axis_index("c")
        pltpu.sync_copy(idx_hbm.at[pl.ds(wid * APW, APW)], idx_v)
        for b in range(NBUF):
            pltpu.async_copy(table_hbm.at[idx_v.at[b]], rows[b], sems[b])

        def group(j, carry):
            for b in range(NBUF):
                a = j * NBUF + b
                pltpu.make_async_copy(
                    table_hbm.at[idx_v.at[a]], rows[b], sems[b]).wait()
                accs = _reduce_rows(rows[b], nreg)
                for k in range(nreg):
                    acc_v[a, pl.ds(k * L, L)] = accs[k]
                nxt = jnp.minimum(a + NBUF, APW - 1)
                pltpu.async_copy(table_hbm.at[idx_v.at[nxt]], rows[b], sems[b])
            return carry

        lax.fori_loop(0, APW // NBUF, group, 0)
        for b in range(NBUF):
            pltpu.make_async_copy(
                table_hbm.at[idx_v.at[0]], rows[b], sems[b]).wait()
        pltpu.sync_copy(acc_v, out_hbm.at[pl.ds(wid * APW, APW)])

    return gsum


def _make_gsum_spmem(h):
    mesh = plsc.VectorSubcoreMesh(core_axis_name="c", subcore_axis_name="s")
    nreg = h // L
    rps = N_PAD // NS

    @functools.partial(
        pl.kernel,
        out_type=jax.ShapeDtypeStruct((N_PAD, h), jnp.float32),
        mesh=mesh,
        compiler_params=pltpu.CompilerParams(use_tc_tiling_on_sc=False),
        scratch_types=[
            pltpu.VMEM((APW, MAX_NB), jnp.int32),
            [pltpu.VMEM((MAX_NB, h), jnp.float32)] * NBUF,
            pltpu.VMEM((AHALF, h), jnp.float32),
            [pltpu.SemaphoreType.DMA] * NBUF,
            pltpu.VMEM_SHARED((N_PAD, h), jnp.float32),
        ],
    )
    def gsum(idx_hbm, table_hbm, out_hbm, idx_v, rows, acc_v, sems, shared):
        sid = lax.axis_index("s")
        wid = sid * NC + lax.axis_index("c")
        pltpu.sync_copy(idx_hbm.at[pl.ds(wid * APW, APW)], idx_v)
        pltpu.sync_copy(table_hbm.at[pl.ds(sid * rps, rps)],
                        shared.at[pl.ds(sid * rps, rps)])
        plsc.subcore_barrier()

        for half in range(2):
            abase = half * AHALF
            for b in range(NBUF):
                pltpu.async_copy(shared.at[idx_v.at[abase + b]], rows[b],
                                 sems[b])

            def group(j, carry, abase=abase):
                for b in range(NBUF):
                    al = j * NBUF + b
                    pltpu.make_async_copy(
                        shared.at[idx_v.at[abase + al]], rows[b],
                        sems[b]).wait()
                    accs = _reduce_rows(rows[b], nreg)
                    for k in range(nreg):
                        acc_v[al, pl.ds(k * L, L)] = accs[k]
                    nxt = abase + jnp.minimum(al + NBUF, AHALF - 1)
                    pltpu.async_copy(shared.at[idx_v.at[nxt]], rows[b],
                                     sems[b])
                return carry

            lax.fori_loop(0, AHALF // NBUF, group, 0)
            for b in range(NBUF):
                pltpu.make_async_copy(
                    shared.at[idx_v.at[0]], rows[b], sems[b]).wait()
            pltpu.sync_copy(acc_v,
                            out_hbm.at[pl.ds(wid * APW + abase, AHALF)])

    return gsum


_gsum_bond = _make_gsum_hbm(BOND_FDIM)
_gsum_msg = _make_gsum_spmem(HIDDEN)


def _tc_init(af, bsum, wit, bi, whbt, bh):
    def body(af_ref, bs_ref, wi_ref, bi_ref, whb_ref, bh_ref, msg0_ref, base_ref):
        inp = jnp.dot(af_ref[...], wi_ref[...],
                      preferred_element_type=jnp.float32) + bi_ref[...]
        msg0_ref[...] = jnp.maximum(inp, 0.0)
        base_ref[...] = inp + jnp.dot(bs_ref[...], whb_ref[...],
                                      preferred_element_type=jnp.float32) + bh_ref[...]

    return pl.pallas_call(
        body,
        out_shape=[jax.ShapeDtypeStruct((N_PAD, HIDDEN), jnp.float32)] * 2,
    )(af, bsum, wit, bi, whbt, bh)


def _tc_update(s, base, whmt):
    def body(s_ref, base_ref, w_ref, out_ref):
        out_ref[...] = jnp.maximum(
            jnp.dot(s_ref[...], w_ref[...],
                    preferred_element_type=jnp.float32) + base_ref[...], 0.0)

    return pl.pallas_call(
        body,
        out_shape=jax.ShapeDtypeStruct((N_PAD, HIDDEN), jnp.float32),
    )(s, base, whmt)


def _tc_out(af, s, woat, womt, bo):
    def body(af_ref, s_ref, wa_ref, wm_ref, bo_ref, out_ref):
        hid = jnp.dot(af_ref[...], wa_ref[...], preferred_element_type=jnp.float32)
        hid = hid + jnp.dot(s_ref[...], wm_ref[...],
                            preferred_element_type=jnp.float32) + bo_ref[...]
        hid = jnp.maximum(hid, 0.0)
        row = lax.broadcasted_iota(jnp.int32, (N_MOLS, N_PAD), 0)
        col = lax.broadcasted_iota(jnp.int32, (N_MOLS, N_PAD), 1)
        pool = jnp.where(col // MOL_SIZE == row, 1.0 / MOL_SIZE, 0.0)
        out_ref[...] = jnp.dot(pool, hid, preferred_element_type=jnp.float32)

    return pl.pallas_call(
        body,
        out_shape=jax.ShapeDtypeStruct((N_MOLS, HIDDEN), jnp.float32),
    )(af, s, woat, womt, bo)


def kernel(atom_features, f_bonds, a2b, a2a, a_scope, W_i, b_i, W_h, b_h, W_o, b_o):
    pad = N_PAD - N_ATOMS
    af_p = jnp.concatenate(
        [atom_features, jnp.zeros((pad, ATOM_FDIM), jnp.float32)], axis=0)
    a2a_p = jnp.concatenate(
        [a2a.astype(jnp.int32), jnp.zeros((pad, MAX_NB), jnp.int32)], axis=0)
    a2b_p = jnp.concatenate(
        [a2b.astype(jnp.int32), jnp.zeros((pad, MAX_NB), jnp.int32)], axis=0)

    wit = W_i.T
    whmt = W_h[:, :HIDDEN].T
    whbt = W_h[:, HIDDEN:].T
    woat = W_o[:, :ATOM_FDIM].T
    womt = W_o[:, ATOM_FDIM:].T
    bi = b_i.reshape(1, HIDDEN)
    bh = b_h.reshape(1, HIDDEN)
    bo = b_o.reshape(1, HIDDEN)

    bond_sum = _gsum_bond(a2b_p, f_bonds)
    msg, base = _tc_init(af_p, bond_sum, wit, bi, whbt, bh)
    for _ in range(DEPTH - 1):
        s = _gsum_msg(a2a_p, msg)
        msg = _tc_update(s, base, whmt)
    s = _gsum_msg(a2a_p, msg)
    return _tc_out(af_p, s, woat, womt, bo)


# speedup vs baseline: 6.5655x; 1.0125x over previous
import functools

import jax
import jax.numpy as jnp
from jax import lax
from jax.experimental import pallas as pl
from jax.experimental.pallas import tpu as pltpu
from jax.experimental.pallas import tpu_sc as plsc

ATOM_FDIM = 128
BOND_FDIM = 16
HIDDEN = 128
DEPTH = 3
N_ATOMS = 10000
MAX_NB = 32
N_MOLS = 100
MOL_SIZE = 100

NC, NS, L = 2, 16, 16
NW = NC * NS
N_PAD = 10240
APW = N_PAD // NW
NBUF = 4
RU = 8
AHALF = APW // 2


def _reduce_rows(rows_ref, nreg):
    def red(ri, accs):
        base = ri * RU
        out = list(accs)
        for rr in range(RU):
            for k in range(nreg):
                out[k] = out[k] + rows_ref[base + rr, pl.ds(k * L, L)]
        return tuple(out)

    zero = jnp.zeros((L,), jnp.float32)
    return lax.fori_loop(0, MAX_NB // RU, red, (zero,) * nreg)


def _make_gsum_hbm(h):
    mesh = plsc.VectorSubcoreMesh(core_axis_name="c", subcore_axis_name="s")
    nreg = h // L

    @functools.partial(
        pl.kernel,
        out_type=jax.ShapeDtypeStruct((N_PAD, h), jnp.float32),
        mesh=mesh,
        compiler_params=pltpu.CompilerParams(use_tc_tiling_on_sc=False),
        scratch_types=[
            pltpu.VMEM((APW, MAX_NB), jnp.int32),
            [pltpu.VMEM((MAX_NB, h), jnp.float32)] * NBUF,
            pltpu.VMEM((APW, h), jnp.float32),
            [pltpu.SemaphoreType.DMA] * NBUF,
        ],
    )
    def gsum(idx_hbm, table_hbm, out_hbm, idx_v, rows, acc_v, sems):
        wid = lax.axis_index("s") * NC + lax.axis_index("c")
        pltpu.sync_copy(idx_hbm.at[pl.ds(wid * APW, APW)], idx_v)
        for b in range(NBUF):
            pltpu.async_copy(table_hbm.at[idx_v.at[b]], rows[b], sems[b])

        def group(j, carry):
            for b in range(NBUF):
                a = j * NBUF + b
                pltpu.make_async_copy(
                    table_hbm.at[idx_v.at[a]], rows[b], sems[b]).wait()
                accs = _reduce_rows(rows[b], nreg)
                for k in range(nreg):
                    acc_v[a, pl.ds(k * L, L)] = accs[k]
                nxt = jnp.minimum(a + NBUF, APW - 1)
                pltpu.async_copy(table_hbm.at[idx_v.at[nxt]], rows[b], sems[b])
            return carry

        lax.fori_loop(0, APW // NBUF, group, 0)
        for b in range(NBUF):
            pltpu.make_async_copy(
                table_hbm.at[idx_v.at[0]], rows[b], sems[b]).wait()
        pltpu.sync_copy(acc_v, out_hbm.at[pl.ds(wid * APW, APW)])

    return gsum


def _make_gsum_spmem(h):
    mesh = plsc.VectorSubcoreMesh(core_axis_name="c", subcore_axis_name="s")
    nreg = h // L
    rps = N_PAD // NS

    @functools.partial(
        pl.kernel,
        out_type=jax.ShapeDtypeStruct((N_PAD, h), jnp.float32),
        mesh=mesh,
        compiler_params=pltpu.CompilerParams(use_tc_tiling_on_sc=False),
        scratch_types=[
            pltpu.VMEM((APW, MAX_NB), jnp.int32),
            [pltpu.VMEM((MAX_NB, h), jnp.float32)] * NBUF,
            pltpu.VMEM((AHALF, h), jnp.float32),
            [pltpu.SemaphoreType.DMA] * NBUF,
            pltpu.VMEM_SHARED((N_PAD, h), jnp.float32),
        ],
    )
    def gsum(idx_hbm, table_hbm, out_hbm, idx_v, rows, acc_v, sems, shared):
        sid = lax.axis_index("s")
        wid = sid * NC + lax.axis_index("c")
        pltpu.sync_copy(idx_hbm.at[pl.ds(wid * APW, APW)], idx_v)
        pltpu.sync_copy(table_hbm.at[pl.ds(sid * rps, rps)],
                        shared.at[pl.ds(sid * rps, rps)])
        plsc.subcore_barrier()

        for half in range(2):
            abase = half * AHALF
            for b in range(NBUF):
                pltpu.async_copy(shared.at[idx_v.at[abase + b]], rows[b],
                                 sems[b])

            def group(j, carry, abase=abase):
                for b in range(NBUF):
                    al = j * NBUF + b
                    pltpu.make_async_copy(
                        shared.at[idx_v.at[abase + al]], rows[b],
                        sems[b]).wait()
                    accs = _reduce_rows(rows[b], nreg)
                    for k in range(nreg):
                        acc_v[al, pl.ds(k * L, L)] = accs[k]
                    nxt = abase + jnp.minimum(al + NBUF, AHALF - 1)
                    pltpu.async_copy(shared.at[idx_v.at[nxt]], rows[b],
                                     sems[b])
                return carry

            lax.fori_loop(0, AHALF // NBUF, group, 0)
            for b in range(NBUF):
                pltpu.make_async_copy(
                    shared.at[idx_v.at[0]], rows[b], sems[b]).wait()
            pltpu.sync_copy(acc_v,
                            out_hbm.at[pl.ds(wid * APW + abase, AHALF)])

    return gsum


_gsum_bond = _make_gsum_hbm(BOND_FDIM)
_gsum_msg = _make_gsum_spmem(HIDDEN)


def _tc_in(af, wit, bi):
    def body(af_ref, wi_ref, bi_ref, msg0_ref, inp_ref):
        inp = jnp.dot(af_ref[...], wi_ref[...],
                      preferred_element_type=jnp.float32) + bi_ref[...]
        msg0_ref[...] = jnp.maximum(inp, 0.0)
        inp_ref[...] = inp

    return pl.pallas_call(
        body,
        out_shape=[jax.ShapeDtypeStruct((N_PAD, HIDDEN), jnp.float32)] * 2,
    )(af, wit, bi)


def _tc_base(inp, bsum, whbt, bh):
    def body(inp_ref, bs_ref, whb_ref, bh_ref, base_ref):
        base_ref[...] = inp_ref[...] + jnp.dot(
            bs_ref[...], whb_ref[...],
            preferred_element_type=jnp.float32) + bh_ref[...]

    return pl.pallas_call(
        body,
        out_shape=jax.ShapeDtypeStruct((N_PAD, HIDDEN), jnp.float32),
    )(inp, bsum, whbt, bh)


def _tc_update(s, base, whmt):
    def body(s_ref, base_ref, w_ref, out_ref):
        out_ref[...] = jnp.maximum(
            jnp.dot(s_ref[...], w_ref[...],
                    preferred_element_type=jnp.float32) + base_ref[...], 0.0)

    return pl.pallas_call(
        body,
        out_shape=jax.ShapeDtypeStruct((N_PAD, HIDDEN), jnp.float32),
    )(s, base, whmt)


def _tc_out(af, s, woat, womt, bo):
    def body(af_ref, s_ref, wa_ref, wm_ref, bo_ref, out_ref):
        hid = jnp.dot(af_ref[...], wa_ref[...], preferred_element_type=jnp.float32)
        hid = hid + jnp.dot(s_ref[...], wm_ref[...],
                            preferred_element_type=jnp.float32) + bo_ref[...]
        hid = jnp.maximum(hid, 0.0)
        row = lax.broadcasted_iota(jnp.int32, (N_MOLS, N_PAD), 0)
        col = lax.broadcasted_iota(jnp.int32, (N_MOLS, N_PAD), 1)
        pool = jnp.where(col // MOL_SIZE == row, 1.0 / MOL_SIZE, 0.0)
        out_ref[...] = jnp.dot(pool, hid, preferred_element_type=jnp.float32)

    return pl.pallas_call(
        body,
        out_shape=jax.ShapeDtypeStruct((N_MOLS, HIDDEN), jnp.float32),
    )(af, s, woat, womt, bo)


def kernel(atom_features, f_bonds, a2b, a2a, a_scope, W_i, b_i, W_h, b_h, W_o, b_o):
    pad = N_PAD - N_ATOMS
    af_p = jnp.concatenate(
        [atom_features, jnp.zeros((pad, ATOM_FDIM), jnp.float32)], axis=0)
    a2a_p = jnp.concatenate(
        [a2a.astype(jnp.int32), jnp.zeros((pad, MAX_NB), jnp.int32)], axis=0)
    a2b_p = jnp.concatenate(
        [a2b.astype(jnp.int32), jnp.zeros((pad, MAX_NB), jnp.int32)], axis=0)

    wit = W_i.T
    whmt = W_h[:, :HIDDEN].T
    whbt = W_h[:, HIDDEN:].T
    woat = W_o[:, :ATOM_FDIM].T
    womt = W_o[:, ATOM_FDIM:].T
    bi = b_i.reshape(1, HIDDEN)
    bh = b_h.reshape(1, HIDDEN)
    bo = b_o.reshape(1, HIDDEN)

    msg, inp = _tc_in(af_p, wit, bi)
    s = _gsum_msg(a2a_p, msg)
    bond_sum = _gsum_bond(a2b_p, f_bonds)
    base = _tc_base(inp, bond_sum, whbt, bh)
    msg = _tc_update(s, base, whmt)
    for _ in range(DEPTH - 2):
        s = _gsum_msg(a2a_p, msg)
        msg = _tc_update(s, base, whmt)
    s = _gsum_msg(a2a_p, msg)
    return _tc_out(af_p, s, woat, womt, bo)
